# baseline (speedup 1.0000x reference)
import functools

import jax
import jax.numpy as jnp
from jax.experimental import pallas as pl

_B, _N, _K = 64, 128, 16
_JETS = 8
_CH = [64, 128, 256]
_NEG = -1e30


def _topk_gather(pts, u, v, k):
    n = pts.shape[0]
    g = jax.lax.dot_general(pts, pts, (((1,), (1,)), ((), ())),
                            preferred_element_type=jnp.float32)
    row = jax.lax.broadcasted_iota(jnp.int32, (n, n), 0)
    col = jax.lax.broadcasted_iota(jnp.int32, (n, n), 1)
    eye = (row == col)
    xx = jnp.sum(jnp.where(eye, g, 0.0), axis=1, keepdims=True)
    sT = 2.0 * g - xx
    sT = jnp.where(eye, _NEG, sT)
    hits = []
    for _ in range(k):
        m = jnp.max(sT, axis=0, keepdims=True)
        hit = (sT == m)
        sT = jnp.where(hit, _NEG, sT)
        hits.append(hit.astype(jnp.float32))
    hit_all = jnp.concatenate(hits, axis=1)
    gath = jax.lax.dot_general(hit_all, u, (((0,), (0,)), ((), ())),
                               preferred_element_type=jnp.float32)
    c = u.shape[1]
    x3 = gath.reshape(k, n, c) + v[None]
    return jax.nn.relu(x3).reshape(k * n, c)


def _fwd_kernel(pts_ref, fts_ref, msk_ref, *args):
    w_refs = args[:-1]
    out_ref = args[-1]
    w = [r[...] for r in w_refs]
    (bn_s, bn_b,
     a0, b0, bnp0, w1t0, bn10, w2t0, bn20, sct0, scbn0,
     a1, b1, bnp1, w1t1, bn11, w2t1, bn21, sct1, scbn1,
     a2, b2, bnp2, w1t2, bn12, w2t2, bn22, sct2, scbn2,
     fc1t, fc1b, fc2t, fc2b) = w

    jj = _JETS
    mask = msk_ref[...].reshape(jj * _N, 1)
    points = pts_ref[...].reshape(jj * _N, 2) * mask
    feats = fts_ref[...].reshape(jj * _N, 7) * mask
    shift = jnp.where(mask == 0.0, 1e9, 0.0)
    counts = [jnp.maximum(jnp.sum(mask[j * _N:(j + 1) * _N]), 1.0)
              for j in range(jj)]

    fts = (feats * bn_s + bn_b) * mask
    pts = points + shift

    layer = ((a0, b0, bnp0, w1t0, bn10, w2t0, bn20, sct0, scbn0),
             (a1, b1, bnp1, w1t1, bn11, w2t1, bn21, sct1, scbn1),
             (a2, b2, bnp2, w1t2, bn12, w2t2, bn22, sct2, scbn2))
    for li, (at, bt, bnb, w1t, b1, w2t, b2, sct, scb) in enumerate(layer):
        u = jnp.dot(fts, bt, preferred_element_type=jnp.float32)
        v = jnp.dot(fts, at, preferred_element_type=jnp.float32) + bnb
        xs = [_topk_gather(pts[j * _N:(j + 1) * _N],
                           u[j * _N:(j + 1) * _N],
                           v[j * _N:(j + 1) * _N], _K) for j in range(jj)]
        x = jnp.concatenate(xs, axis=0)
        x = jnp.dot(x, w1t, preferred_element_type=jnp.float32)
        x = jax.nn.relu(x + b1)
        x = jnp.dot(x, w2t, preferred_element_type=jnp.float32)
        x = jax.nn.relu(x + b2)
        means = []
        for j in range(jj):
            x3 = x[j * _K * _N:(j + 1) * _K * _N].reshape(_K, _N, -1)
            means.append(jnp.sum(x3, axis=0) * (1.0 / _K))
        mean = jnp.concatenate(means, axis=0)
        sc = jnp.dot(fts, sct, preferred_element_type=jnp.float32) + scb
        fts = jax.nn.relu(sc + mean) * mask
        pts = fts + shift

    pooled = jnp.concatenate(
        [jnp.sum(fts[j * _N:(j + 1) * _N], axis=0, keepdims=True) / counts[j]
         for j in range(jj)], axis=0)
    h = jax.nn.relu(jnp.dot(pooled, fc1t,
                            preferred_element_type=jnp.float32) + fc1b)
    out = jnp.dot(h, fc2t, preferred_element_type=jnp.float32) + fc2b
    out_ref[...] = out.reshape(out_ref.shape)


@jax.jit
def _run(pts_r, fts_r, msk_r, weights):
    full = lambda shp: pl.BlockSpec(shp, lambda b: (0,) * len(shp))
    in_specs = [
        pl.BlockSpec((_JETS, _N, 2), lambda b: (b, 0, 0)),
        pl.BlockSpec((_JETS, _N, 7), lambda b: (b, 0, 0)),
        pl.BlockSpec((_JETS, _N, 1), lambda b: (b, 0, 0)),
    ] + [full(w.shape) for w in weights]
    return pl.pallas_call(
        _fwd_kernel,
        grid=(_B // _JETS,),
        in_specs=in_specs,
        out_specs=pl.BlockSpec((_JETS, 1, 10), lambda b: (b, 0, 0)),
        out_shape=jax.ShapeDtypeStruct((_B, 1, 10), jnp.float32),
    )(pts_r, fts_r, msk_r, *weights)


def kernel(points, features, mask, params):
    p = params
    f32 = jnp.float32

    def rowv(x):
        return x.reshape(1, -1).astype(f32)

    weights = [rowv(p['bn_fts_s']), rowv(p['bn_fts_b'])]
    for l in range(3):
        w0 = p['ec%d_w0' % l]
        cin = w0.shape[1] // 2
        w0a, w0b = w0[:, :cin], w0[:, cin:]
        s0 = p['ec%d_bn0_s' % l][None, :].astype(f32)
        s1 = p['ec%d_bn1_s' % l][None, :].astype(f32)
        s2 = p['ec%d_bn2_s' % l][None, :].astype(f32)
        ssc = p['ec%d_sc_s' % l][None, :].astype(f32)
        weights += [
            (w0a - w0b).T.astype(f32) * s0,
            w0b.T.astype(f32) * s0,
            rowv(p['ec%d_bn0_b' % l]),
            p['ec%d_w1' % l].T.astype(f32) * s1,
            rowv(p['ec%d_bn1_b' % l]),
            p['ec%d_w2' % l].T.astype(f32) * s2,
            rowv(p['ec%d_bn2_b' % l]),
            p['ec%d_sc_w' % l].T.astype(f32) * ssc,
            rowv(p['ec%d_sc_b' % l]),
        ]
    weights += [p['fc1_w'].T.astype(f32), p['fc1_b'].reshape(1, -1).astype(f32),
                p['fc2_w'].T.astype(f32), p['fc2_b'].reshape(1, -1).astype(f32)]

    pts_r = jnp.transpose(points, (0, 2, 1))
    fts_r = jnp.transpose(features, (0, 2, 1))
    msk_r = jnp.transpose(mask, (0, 2, 1))
    out = _run(pts_r, fts_r, msk_r, tuple(weights))
    return out.reshape(_B, 10)


# speedup vs baseline: 1.0014x; 1.0014x over previous
import jax
import jax.numpy as jnp
from jax.experimental import pallas as pl

_B, _N, _K = 64, 128, 16
_JETS = 4
_NEG = -1e30


def _topk_gather(pts, u, v, k):
    n = pts.shape[0]
    g = jax.lax.dot_general(pts, pts, (((1,), (1,)), ((), ())),
                            preferred_element_type=jnp.float32)
    row = jax.lax.broadcasted_iota(jnp.int32, (n, n), 0)
    col = jax.lax.broadcasted_iota(jnp.int32, (n, n), 1)
    eye = (row == col)
    xx = jnp.sum(jnp.where(eye, g, 0.0), axis=1, keepdims=True)
    sT = 2.0 * g - xx
    sT = jnp.where(eye, _NEG, sT)
    hits = []
    for _ in range(k):
        m = jnp.max(sT, axis=0, keepdims=True)
        hit = (sT == m)
        sT = jnp.where(hit, _NEG, sT)
        hits.append(hit.astype(jnp.float32))
    hit_all = jnp.concatenate(hits, axis=1)
    gath = jax.lax.dot_general(hit_all, u, (((0,), (0,)), ((), ())),
                               preferred_element_type=jnp.float32)
    c = u.shape[1]
    x3 = gath.reshape(k, n, c) + v[None]
    return jax.nn.relu(x3).reshape(k * n, c)


def _fwd_kernel(pts_ref, fts_ref, msk_ref, *args):
    w_refs = args[:-1]
    out_ref = args[-1]
    w = [r[...] for r in w_refs]
    (bn_s, bn_b,
     a0, b0, bnp0, w1t0, bn10, w2t0, bn20, sct0, scbn0,
     a1, b1, bnp1, w1t1, bn11, w2t1, bn21, sct1, scbn1,
     a2, b2, bnp2, w1t2, bn12, w2t2, bn22, sct2, scbn2,
     fc1t, fc1b, fc2t, fc2b) = w

    jj = _JETS
    mask = msk_ref[...].reshape(jj * _N, 1)
    points = pts_ref[...].reshape(jj * _N, 2) * mask
    feats = fts_ref[...].reshape(jj * _N, 7) * mask
    shift = jnp.where(mask == 0.0, 1e9, 0.0)
    counts = [jnp.maximum(jnp.sum(mask[j * _N:(j + 1) * _N]), 1.0)
              for j in range(jj)]

    fts = (feats * bn_s + bn_b) * mask
    pts = points + shift

    layer = ((a0, b0, bnp0, w1t0, bn10, w2t0, bn20, sct0, scbn0),
             (a1, b1, bnp1, w1t1, bn11, w2t1, bn21, sct1, scbn1),
             (a2, b2, bnp2, w1t2, bn12, w2t2, bn22, sct2, scbn2))
    for (at, bt, bnb, w1t, b1, w2t, b2, sct, scb) in layer:
        u = jnp.dot(fts, bt, preferred_element_type=jnp.float32)
        v = jnp.dot(fts, at, preferred_element_type=jnp.float32) + bnb
        xs = [_topk_gather(pts[j * _N:(j + 1) * _N],
                           u[j * _N:(j + 1) * _N],
                           v[j * _N:(j + 1) * _N], _K) for j in range(jj)]
        x = jnp.concatenate(xs, axis=0)
        x = jnp.dot(x, w1t, preferred_element_type=jnp.float32)
        x = jax.nn.relu(x + b1)
        x = jnp.dot(x, w2t, preferred_element_type=jnp.float32)
        x = jax.nn.relu(x + b2)
        means = []
        for j in range(jj):
            x3 = x[j * _K * _N:(j + 1) * _K * _N].reshape(_K, _N, -1)
            means.append(jnp.sum(x3, axis=0) * (1.0 / _K))
        mean = jnp.concatenate(means, axis=0)
        sc = jnp.dot(fts, sct, preferred_element_type=jnp.float32) + scb
        fts = jax.nn.relu(sc + mean) * mask
        pts = fts + shift

    pooled = jnp.concatenate(
        [jnp.sum(fts[j * _N:(j + 1) * _N], axis=0, keepdims=True) / counts[j]
         for j in range(jj)], axis=0)
    h = jax.nn.relu(jnp.dot(pooled, fc1t,
                            preferred_element_type=jnp.float32) + fc1b)
    out = jnp.dot(h, fc2t, preferred_element_type=jnp.float32) + fc2b
    out_ref[...] = out.reshape(out_ref.shape)


@jax.jit
def _run(pts_r, fts_r, msk_r, weights):
    full = lambda shp: pl.BlockSpec(shp, lambda b: (0,) * len(shp))
    in_specs = [
        pl.BlockSpec((_JETS, _N, 2), lambda b: (b, 0, 0)),
        pl.BlockSpec((_JETS, _N, 7), lambda b: (b, 0, 0)),
        pl.BlockSpec((_JETS, _N, 1), lambda b: (b, 0, 0)),
    ] + [full(w.shape) for w in weights]
    return pl.pallas_call(
        _fwd_kernel,
        grid=(_B // _JETS,),
        in_specs=in_specs,
        out_specs=pl.BlockSpec((_JETS, 1, 10), lambda b: (b, 0, 0)),
        out_shape=jax.ShapeDtypeStruct((_B, 1, 10), jnp.float32),
    )(pts_r, fts_r, msk_r, *weights)


def kernel(points, features, mask, params):
    p = params
    f32 = jnp.float32

    def rowv(x):
        return x.reshape(1, -1).astype(f32)

    weights = [rowv(p['bn_fts_s']), rowv(p['bn_fts_b'])]
    for l in range(3):
        w0 = p['ec%d_w0' % l]
        cin = w0.shape[1] // 2
        w0a, w0b = w0[:, :cin], w0[:, cin:]
        s0 = p['ec%d_bn0_s' % l][None, :].astype(f32)
        s1 = p['ec%d_bn1_s' % l][None, :].astype(f32)
        s2 = p['ec%d_bn2_s' % l][None, :].astype(f32)
        ssc = p['ec%d_sc_s' % l][None, :].astype(f32)
        weights += [
            (w0a - w0b).T.astype(f32) * s0,
            w0b.T.astype(f32) * s0,
            rowv(p['ec%d_bn0_b' % l]),
            p['ec%d_w1' % l].T.astype(f32) * s1,
            rowv(p['ec%d_bn1_b' % l]),
            p['ec%d_w2' % l].T.astype(f32) * s2,
            rowv(p['ec%d_bn2_b' % l]),
            p['ec%d_sc_w' % l].T.astype(f32) * ssc,
            rowv(p['ec%d_sc_b' % l]),
        ]
    weights += [p['fc1_w'].T.astype(f32), p['fc1_b'].reshape(1, -1).astype(f32),
                p['fc2_w'].T.astype(f32), p['fc2_b'].reshape(1, -1).astype(f32)]

    pts_r = jnp.transpose(points, (0, 2, 1))
    fts_r = jnp.transpose(features, (0, 2, 1))
    msk_r = jnp.transpose(mask, (0, 2, 1))
    out = _run(pts_r, fts_r, msk_r, tuple(weights))
    return out.reshape(_B, 10)
